# baseline (speedup 1.0000x reference)
import functools

import jax
import jax.numpy as jnp
from jax import lax
from jax.experimental import pallas as pl
from jax.experimental.pallas import tpu as pltpu
from jax.experimental.pallas import tpu_sc as plsc

B, N, VOCAB, EMB = 1024, 50, 1000, 64
NT = 18
NS = N - NT
LG = 8
BG = B // LG
VC = 200
NCH = VOCAB // VC


def _onehot_matmul(idx_of_lg, wt, out_ref):
    for lg in range(LG):
        oh = (
            lax.broadcasted_iota(jnp.int32, (VOCAB, BG), 0)
            == idx_of_lg(lg)[None, :]
        ).astype(jnp.float32)
        out_ref[0, :, lg * BG : (lg + 1) * BG] = jax.lax.dot(
            wt, oh, preferred_element_type=jnp.float32
        )


def _tc1_block(x_ref, w_ref, out_ref):
    xb = x_ref[0]
    mx = jnp.max(xb, axis=0, keepdims=True)
    iota = lax.broadcasted_iota(jnp.int32, (VOCAB, B), 0)
    idx = jnp.min(jnp.where(xb == mx, iota, VOCAB), axis=0)
    _onehot_matmul(lambda lg: idx[lg * BG : (lg + 1) * BG], w_ref[...], out_ref)


def _tc1(xt, wt):
    return pl.pallas_call(
        _tc1_block,
        grid=(NT,),
        in_specs=[
            pl.BlockSpec((1, VOCAB, B), lambda i: (i, 0, 0)),
            pl.BlockSpec((EMB, VOCAB), lambda i: (0, 0)),
        ],
        out_specs=pl.BlockSpec((1, EMB, B), lambda i: (i, 0, 0)),
        out_shape=jax.ShapeDtypeStruct((N, EMB, B), jnp.float32),
    )(xt, wt)


def _sc_tokens(xt):
    info = plsc.get_sparse_core_info()
    nw = info.num_cores * info.num_subcores
    assert nw == NS
    mesh = plsc.VectorSubcoreMesh(core_axis_name="c", subcore_axis_name="s")

    @functools.partial(
        pl.kernel,
        mesh=mesh,
        out_type=jax.ShapeDtypeStruct((NS, LG, BG), jnp.int32),
        scratch_types=[
            pltpu.VMEM((2, VC, BG), jnp.float32),
            pltpu.VMEM((LG, BG), jnp.int32),
            pltpu.SemaphoreType.DMA,
            pltpu.SemaphoreType.DMA,
        ],
        compiler_params=pltpu.CompilerParams(needs_layout_passes=False),
    )
    def sc_argmax(x_hbm, tok_hbm, xbuf, tokbuf, d0, d1):
        wid = lax.axis_index("s") * info.num_cores + lax.axis_index("c")
        n = NT + wid
        sems = [d0, d1]
        neg_inf = jnp.full((16,), -jnp.inf, jnp.float32)
        zeros = jnp.zeros((16,), jnp.int32)

        def issue(k):
            lg, c = k // NCH, k % NCH
            return pltpu.async_copy(
                x_hbm.at[n, pl.ds(c * VC, VC), pl.ds(lg * BG, BG)],
                xbuf.at[k & 1],
                sems[k & 1],
            )

        total = LG * NCH
        inflight = [issue(0), issue(1)]
        ms = mis = None
        for k in range(total):
            lg, c = k // NCH, k % NCH
            p = k & 1
            inflight[p].wait()
            if c == 0:
                ms = [neg_inf] * 8
                mis = [zeros] * 8

            def chunk_body(i, carry, p=p):
                gid, cms, cmis = carry
                nms, nmis = list(cms), list(cmis)
                for u in range(2):
                    gu = gid + u
                    for g in range(8):
                        val = xbuf[p, 2 * i + u, pl.ds(g * 16, 16)]
                        gt = val > nms[g]
                        nms[g] = jnp.where(gt, val, nms[g])
                        nmis[g] = jnp.where(gt, gu, nmis[g])
                return gid + 2, tuple(nms), tuple(nmis)

            gid0 = jnp.full((16,), c * VC, jnp.int32)
            _, ms, mis = lax.fori_loop(
                0, VC // 2, chunk_body, (gid0, tuple(ms), tuple(mis))
            )
            if k + 2 < total:
                inflight[p] = issue(k + 2)
            if c == NCH - 1:
                for g in range(8):
                    tokbuf[lg, pl.ds(g * 16, 16)] = mis[g]
        pltpu.sync_copy(tokbuf, tok_hbm.at[wid])

    return sc_argmax(xt)


def _tc2_block(tok_ref, w_ref, carry_ref, out_ref):
    del carry_ref
    idx2 = tok_ref[0]
    _onehot_matmul(lambda lg: idx2[lg], w_ref[...], out_ref)


def _tc2(toks, wt, out_carry):
    return pl.pallas_call(
        _tc2_block,
        grid=(NS,),
        in_specs=[
            pl.BlockSpec((1, LG, BG), lambda i: (i, 0, 0)),
            pl.BlockSpec((EMB, VOCAB), lambda i: (0, 0)),
            pl.BlockSpec(memory_space=pltpu.MemorySpace.HBM),
        ],
        out_specs=pl.BlockSpec((1, EMB, B), lambda i: (NT + i, 0, 0)),
        out_shape=jax.ShapeDtypeStruct((N, EMB, B), jnp.float32),
        input_output_aliases={2: 0},
    )(toks, wt, out_carry)


def kernel(x, W):
    xt = jnp.transpose(x, (1, 2, 0))
    wt = jnp.transpose(W, (1, 0))
    out_tc = _tc1(xt, wt)
    toks = _sc_tokens(xt)
    out_t = _tc2(toks, wt, out_tc)
    return jnp.transpose(out_t, (2, 0, 1))


# speedup vs baseline: 1.0622x; 1.0622x over previous
import functools

import jax
import jax.numpy as jnp
from jax import lax
from jax.experimental import pallas as pl
from jax.experimental.pallas import tpu as pltpu
from jax.experimental.pallas import tpu_sc as plsc

B, N, VOCAB, EMB = 1024, 50, 1000, 64
NT = 26
NS = N - NT
LG = 8
BG = B // LG
VC = 200
NCH = VOCAB // VC


def _onehot_matmul(idx_of_lg, wt, out_ref):
    for lg in range(LG):
        oh = (
            lax.broadcasted_iota(jnp.int32, (VOCAB, BG), 0)
            == idx_of_lg(lg)[None, :]
        ).astype(jnp.float32)
        out_ref[0, :, lg * BG : (lg + 1) * BG] = jax.lax.dot(
            wt, oh, preferred_element_type=jnp.float32
        )


def _tc1_block(x_ref, w_ref, out_ref):
    xb = x_ref[0]
    mx = jnp.max(xb, axis=0, keepdims=True)
    iota = lax.broadcasted_iota(jnp.int32, (VOCAB, B), 0)
    idx = jnp.min(jnp.where(xb == mx, iota, VOCAB), axis=0)
    _onehot_matmul(lambda lg: idx[lg * BG : (lg + 1) * BG], w_ref[...], out_ref)


def _tc1(xt, wt):
    return pl.pallas_call(
        _tc1_block,
        grid=(NT,),
        in_specs=[
            pl.BlockSpec((1, VOCAB, B), lambda i: (i, 0, 0)),
            pl.BlockSpec((EMB, VOCAB), lambda i: (0, 0)),
        ],
        out_specs=pl.BlockSpec((1, EMB, B), lambda i: (i, 0, 0)),
        out_shape=jax.ShapeDtypeStruct((N, EMB, B), jnp.float32),
    )(xt, wt)


def _sc_tokens(xt):
    info = plsc.get_sparse_core_info()
    mesh = plsc.VectorSubcoreMesh(core_axis_name="c", subcore_axis_name="s")

    @functools.partial(
        pl.kernel,
        mesh=mesh,
        out_type=jax.ShapeDtypeStruct((NS, LG, BG), jnp.int32),
        scratch_types=[
            pltpu.VMEM((2, VC, BG), jnp.float32),
            pltpu.VMEM((LG, BG), jnp.int32),
            pltpu.SemaphoreType.DMA,
            pltpu.SemaphoreType.DMA,
        ],
        compiler_params=pltpu.CompilerParams(needs_layout_passes=False),
    )
    def sc_argmax(x_hbm, tok_hbm, xbuf, tokbuf, d0, d1):
        wid = lax.axis_index("s") * info.num_cores + lax.axis_index("c")

        @pl.when(wid < NS)
        def _body():
            _sc_row(x_hbm, tok_hbm, xbuf, tokbuf, d0, d1, wid)

    return sc_argmax(xt)


def _sc_row(x_hbm, tok_hbm, xbuf, tokbuf, d0, d1, wid):
        n = NT + wid
        sems = [d0, d1]
        neg_inf = jnp.full((16,), -jnp.inf, jnp.float32)
        zeros = jnp.zeros((16,), jnp.int32)

        def issue(k):
            lg, c = k // NCH, k % NCH
            return pltpu.async_copy(
                x_hbm.at[n, pl.ds(c * VC, VC), pl.ds(lg * BG, BG)],
                xbuf.at[k & 1],
                sems[k & 1],
            )

        total = LG * NCH
        inflight = [issue(0), issue(1)]
        ms = mis = None
        for k in range(total):
            lg, c = k // NCH, k % NCH
            p = k & 1
            inflight[p].wait()
            if c == 0:
                ms = [neg_inf] * 8
                mis = [zeros] * 8

            def chunk_body(i, carry, p=p):
                gid, cms, cmis = carry
                nms, nmis = list(cms), list(cmis)
                for u in range(2):
                    gu = gid + u
                    for g in range(8):
                        val = xbuf[p, 2 * i + u, pl.ds(g * 16, 16)]
                        gt = val > nms[g]
                        nms[g] = jnp.where(gt, val, nms[g])
                        nmis[g] = jnp.where(gt, gu, nmis[g])
                return gid + 2, tuple(nms), tuple(nmis)

            gid0 = jnp.full((16,), c * VC, jnp.int32)
            _, ms, mis = lax.fori_loop(
                0, VC // 2, chunk_body, (gid0, tuple(ms), tuple(mis))
            )
            if k + 2 < total:
                inflight[p] = issue(k + 2)
            if c == NCH - 1:
                for g in range(8):
                    tokbuf[lg, pl.ds(g * 16, 16)] = mis[g]
        pltpu.sync_copy(tokbuf, tok_hbm.at[wid])


def _tc2_block(tok_ref, w_ref, carry_ref, out_ref):
    del carry_ref
    idx2 = tok_ref[0]
    _onehot_matmul(lambda lg: idx2[lg], w_ref[...], out_ref)


def _tc2(toks, wt, out_carry):
    return pl.pallas_call(
        _tc2_block,
        grid=(NS,),
        in_specs=[
            pl.BlockSpec((1, LG, BG), lambda i: (i, 0, 0)),
            pl.BlockSpec((EMB, VOCAB), lambda i: (0, 0)),
            pl.BlockSpec(memory_space=pltpu.MemorySpace.HBM),
        ],
        out_specs=pl.BlockSpec((1, EMB, B), lambda i: (NT + i, 0, 0)),
        out_shape=jax.ShapeDtypeStruct((N, EMB, B), jnp.float32),
        input_output_aliases={2: 0},
    )(toks, wt, out_carry)


def kernel(x, W):
    xt = jnp.transpose(x, (1, 2, 0))
    wt = jnp.transpose(W, (1, 0))
    out_tc = _tc1(xt, wt)
    toks = _sc_tokens(xt)
    out_t = _tc2(toks, wt, out_tc)
    return jnp.transpose(out_t, (2, 0, 1))


# speedup vs baseline: 1.1229x; 1.0571x over previous
import functools

import jax
import jax.numpy as jnp
from jax import lax
from jax.experimental import pallas as pl
from jax.experimental.pallas import tpu as pltpu
from jax.experimental.pallas import tpu_sc as plsc

B, N, VOCAB, EMB = 1024, 50, 1000, 64
NT = 34
NS = N - NT
HL = 4
LG = 8
BG = B // LG
VC = 200
NCH = VOCAB // VC


def _onehot_matmul(idx_of_lg, wt, out_ref):
    for lg in range(LG):
        oh = (
            lax.broadcasted_iota(jnp.int32, (VOCAB, BG), 0)
            == idx_of_lg(lg)[None, :]
        ).astype(jnp.float32)
        out_ref[0, :, lg * BG : (lg + 1) * BG] = jax.lax.dot(
            wt, oh, preferred_element_type=jnp.float32
        )


def _tc1_block(x_ref, w_ref, out_ref):
    xb = x_ref[0]
    mx = jnp.max(xb, axis=0, keepdims=True)
    iota = lax.broadcasted_iota(jnp.int32, (VOCAB, B), 0)
    idx = jnp.min(jnp.where(xb == mx, iota, VOCAB), axis=0)
    _onehot_matmul(lambda lg: idx[lg * BG : (lg + 1) * BG], w_ref[...], out_ref)


def _tc1(xt, wt):
    return pl.pallas_call(
        _tc1_block,
        grid=(NT,),
        in_specs=[
            pl.BlockSpec((1, VOCAB, B), lambda i: (i, 0, 0)),
            pl.BlockSpec((EMB, VOCAB), lambda i: (0, 0)),
        ],
        out_specs=pl.BlockSpec((1, EMB, B), lambda i: (i, 0, 0)),
        out_shape=jax.ShapeDtypeStruct((N, EMB, B), jnp.float32),
    )(xt, wt)


def _sc_tokens(xt):
    info = plsc.get_sparse_core_info()
    mesh = plsc.VectorSubcoreMesh(core_axis_name="c", subcore_axis_name="s")

    @functools.partial(
        pl.kernel,
        mesh=mesh,
        out_type=jax.ShapeDtypeStruct((NS, 2, HL, BG), jnp.int32),
        scratch_types=[
            pltpu.VMEM((2, VC, BG), jnp.float32),
            pltpu.VMEM((HL, BG), jnp.int32),
            pltpu.SemaphoreType.DMA,
            pltpu.SemaphoreType.DMA,
        ],
        compiler_params=pltpu.CompilerParams(needs_layout_passes=False),
    )
    def sc_argmax(x_hbm, tok_hbm, xbuf, tokbuf, d0, d1):
        wid = lax.axis_index("s") * info.num_cores + lax.axis_index("c")

        _sc_row(x_hbm, tok_hbm, xbuf, tokbuf, d0, d1, wid)

    return sc_argmax(xt)


def _sc_row(x_hbm, tok_hbm, xbuf, tokbuf, d0, d1, wid):
        n = NT + wid // 2
        h = wid % 2
        sems = [d0, d1]
        neg_inf = jnp.full((16,), -jnp.inf, jnp.float32)
        zeros = jnp.zeros((16,), jnp.int32)

        def issue(k):
            lg, c = k // NCH, k % NCH
            return pltpu.async_copy(
                x_hbm.at[n, pl.ds(c * VC, VC), pl.ds((h * HL + lg) * BG, BG)],
                xbuf.at[k & 1],
                sems[k & 1],
            )

        total = HL * NCH
        inflight = [issue(0), issue(1)]
        ms = mis = None
        for k in range(total):
            lg, c = k // NCH, k % NCH
            p = k & 1
            inflight[p].wait()
            if c == 0:
                ms = [neg_inf] * 8
                mis = [zeros] * 8

            def chunk_body(i, carry, p=p):
                gid, cms, cmis = carry
                nms, nmis = list(cms), list(cmis)
                for u in range(2):
                    gu = gid + u
                    for g in range(8):
                        val = xbuf[p, 2 * i + u, pl.ds(g * 16, 16)]
                        gt = val > nms[g]
                        nms[g] = jnp.where(gt, val, nms[g])
                        nmis[g] = jnp.where(gt, gu, nmis[g])
                return gid + 2, tuple(nms), tuple(nmis)

            gid0 = jnp.full((16,), c * VC, jnp.int32)
            _, ms, mis = lax.fori_loop(
                0, VC // 2, chunk_body, (gid0, tuple(ms), tuple(mis))
            )
            if k + 2 < total:
                inflight[p] = issue(k + 2)
            if c == NCH - 1:
                for g in range(8):
                    tokbuf[lg, pl.ds(g * 16, 16)] = mis[g]
        pltpu.sync_copy(tokbuf, tok_hbm.at[wid // 2, h])


def _tc2_block(tok_ref, w_ref, carry_ref, out_ref):
    del carry_ref
    idx2 = tok_ref[0]
    _onehot_matmul(lambda lg: idx2[lg // HL, lg % HL], w_ref[...], out_ref)


def _tc2(toks, wt, out_carry):
    return pl.pallas_call(
        _tc2_block,
        grid=(NS,),
        in_specs=[
            pl.BlockSpec((1, 2, HL, BG), lambda i: (i, 0, 0, 0)),
            pl.BlockSpec((EMB, VOCAB), lambda i: (0, 0)),
            pl.BlockSpec(memory_space=pltpu.MemorySpace.HBM),
        ],
        out_specs=pl.BlockSpec((1, EMB, B), lambda i: (NT + i, 0, 0)),
        out_shape=jax.ShapeDtypeStruct((N, EMB, B), jnp.float32),
        input_output_aliases={2: 0},
    )(toks, wt, out_carry)


def kernel(x, W):
    xt = jnp.transpose(x, (1, 2, 0))
    wt = jnp.transpose(W, (1, 0))
    out_tc = _tc1(xt, wt)
    toks = _sc_tokens(xt)
    out_t = _tc2(toks, wt, out_tc)
    return jnp.transpose(out_t, (2, 0, 1))
